# baseline (speedup 1.0000x reference)
import functools

import jax
import jax.numpy as jnp
from jax import lax
from jax.experimental import pallas as pl
from jax.experimental.pallas import tpu as pltpu
from jax.experimental.pallas import tpu_sc as plsc

VOCAB = 51200
EMBED_DIM = 1024
NUM_CORES = 2
NUM_SUBCORES = 16
NW = NUM_CORES * NUM_SUBCORES
B_TOTAL = 4 * 2048
B_PER_W = B_TOTAL // NW
CHUNK = 16
N_CHUNKS = B_PER_W // CHUNK
NBUF = 6

_mesh = plsc.VectorSubcoreMesh(core_axis_name="c", subcore_axis_name="s")


@functools.partial(
    pl.kernel,
    out_type=jax.ShapeDtypeStruct((B_TOTAL, EMBED_DIM), jnp.float32),
    mesh=_mesh,
    scratch_types=[
        pltpu.VMEM((N_CHUNKS, CHUNK), jnp.int32),
    ] + [pltpu.VMEM((CHUNK, EMBED_DIM), jnp.float32)] * NBUF
      + [pltpu.SemaphoreType.DMA] * (2 * NBUF),
)
def _embedding_gather(ids_hbm, table_hbm, out_hbm, idx_v, *rest):
    bufs = list(rest[:NBUF])
    gsems = list(rest[NBUF:2 * NBUF])
    ssems = list(rest[2 * NBUF:3 * NBUF])
    wid = lax.axis_index("s") * NUM_CORES + lax.axis_index("c")
    base = wid * B_PER_W

    pltpu.sync_copy(ids_hbm.at[wid], idx_v)

    gathers = [None] * N_CHUNKS
    stores = [None] * N_CHUNKS

    def start_gather(j):
        b = j % NBUF
        gathers[j] = pltpu.async_copy(
            table_hbm.at[idx_v.at[j]], bufs[b], gsems[b])

    def start_store(j):
        b = j % NBUF
        stores[j] = pltpu.async_copy(
            bufs[b], out_hbm.at[pl.ds(base + j * CHUNK, CHUNK)], ssems[b])

    for j in range(NBUF):
        start_gather(j)
    for j in range(N_CHUNKS):
        if j >= 1:
            k = j - 1 + NBUF
            if k < N_CHUNKS:
                stores[k - NBUF].wait()
                start_gather(k)
        gathers[j].wait()
        start_store(j)
    for j in range(N_CHUNKS - NBUF, N_CHUNKS):
        stores[j].wait()


def kernel(input_ids, wte):
    ids = input_ids.reshape(-1).astype(jnp.int32)
    ids3 = ids.reshape(NW, N_CHUNKS, CHUNK)
    out = _embedding_gather(ids3, wte)
    return out.reshape(input_ids.shape + (EMBED_DIM,))


# speedup vs baseline: 1.0054x; 1.0054x over previous
import functools

import jax
import jax.numpy as jnp
from jax import lax
from jax.experimental import pallas as pl
from jax.experimental.pallas import tpu as pltpu
from jax.experimental.pallas import tpu_sc as plsc

VOCAB = 51200
EMBED_DIM = 1024
NUM_CORES = 2
NUM_SUBCORES = 16
NW = NUM_CORES * NUM_SUBCORES
B_TOTAL = 4 * 2048
B_PER_W = B_TOTAL // NW
CHUNK = 16
N_CHUNKS = B_PER_W // CHUNK
NBUF = 6

_mesh = plsc.VectorSubcoreMesh(core_axis_name="c", subcore_axis_name="s")


@functools.partial(
    pl.kernel,
    out_type=jax.ShapeDtypeStruct((4, 2048, EMBED_DIM), jnp.float32),
    mesh=_mesh,
    scratch_types=[
        pltpu.VMEM((N_CHUNKS, CHUNK), jnp.int32),
    ] + [pltpu.VMEM((CHUNK, EMBED_DIM), jnp.float32)] * NBUF
      + [pltpu.SemaphoreType.DMA] * (2 * NBUF),
)
def _embedding_gather(ids_hbm, table_hbm, out_hbm, idx_v, *rest):
    bufs = list(rest[:NBUF])
    gsems = list(rest[NBUF:2 * NBUF])
    ssems = list(rest[2 * NBUF:3 * NBUF])
    wid = lax.axis_index("s") * NUM_CORES + lax.axis_index("c")
    bi = wid // (2048 // B_PER_W)
    seq0 = (wid % (2048 // B_PER_W)) * B_PER_W

    pltpu.sync_copy(ids_hbm.at[wid], idx_v)

    gathers = [None] * N_CHUNKS
    stores = [None] * N_CHUNKS

    def start_gather(j):
        b = j % NBUF
        gathers[j] = pltpu.async_copy(
            table_hbm.at[idx_v.at[j]], bufs[b], gsems[b])

    def start_store(j):
        b = j % NBUF
        stores[j] = pltpu.async_copy(
            bufs[b], out_hbm.at[bi, pl.ds(seq0 + j * CHUNK, CHUNK)], ssems[b])

    for j in range(NBUF):
        start_gather(j)
    for j in range(N_CHUNKS):
        if j >= 1:
            k = j - 1 + NBUF
            if k < N_CHUNKS:
                stores[k - NBUF].wait()
                start_gather(k)
        gathers[j].wait()
        start_store(j)
    for j in range(N_CHUNKS - NBUF, N_CHUNKS):
        stores[j].wait()


def kernel(input_ids, wte):
    ids = input_ids.reshape(-1).astype(jnp.int32)
    ids3 = ids.reshape(NW, N_CHUNKS, CHUNK)
    return _embedding_gather(ids3, wte)
